# Initial kernel scaffold; baseline (speedup 1.0000x reference)
import functools

import jax
import jax.numpy as jnp
from jax import lax
from jax.experimental import pallas as pl
from jax.experimental.pallas import tpu as pltpu
from jax.experimental.pallas import tpu_sc as plsc

NC = 2
NS = 16
NW = NC * NS
CHUNK = 128
BLK = 1024


def _round_up(a, b):
    return (a + b - 1) // b * b


def _sc_degree(dst3, n_pad):
    nchunks = dst3.shape[1]
    rpt = n_pad // NS

    mesh = plsc.VectorSubcoreMesh(core_axis_name="c", subcore_axis_name="s")

    @functools.partial(
        pl.kernel,
        out_type=jax.ShapeDtypeStruct((NC, n_pad), jnp.float32),
        mesh=mesh,
        scratch_types=[
            pltpu.VMEM((nchunks, CHUNK), jnp.int32),
            pltpu.VMEM((CHUNK,), jnp.float32),
            pltpu.VMEM((rpt,), jnp.float32),
            pltpu.VMEM_SHARED((n_pad,), jnp.float32),
        ],
    )
    def k(dst_hbm, deg_hbm, dst_v, ones_v, stage_v, deg_sh):
        c = lax.axis_index("c")
        s = lax.axis_index("s")
        slab = c * NS + s
        ones = jnp.ones((16,), jnp.float32)
        zeros = jnp.zeros((16,), jnp.float32)
        for u in range(CHUNK // 16):
            ones_v[pl.ds(u * 16, 16)] = ones

        def zbody(r, _):
            stage_v[pl.ds(r * 16, 16)] = zeros
            return ()
        lax.fori_loop(0, rpt // 16, zbody, ())
        pltpu.sync_copy(stage_v, deg_sh.at[pl.ds(s * rpt, rpt)])
        plsc.subcore_barrier()

        pltpu.sync_copy(dst_hbm.at[slab], dst_v)

        def body(j, _):
            pltpu.sync_copy(ones_v, deg_sh.at[dst_v.at[j]], add=True)
            return ()
        lax.fori_loop(0, nchunks, body, ())
        plsc.subcore_barrier()

        pltpu.sync_copy(deg_sh.at[pl.ds(s * rpt, rpt)], stage_v)
        pltpu.sync_copy(stage_v, deg_hbm.at[c, pl.ds(s * rpt, rpt)])

    return k(dst3)


def _sc_scatter(g, src3, dst3, n_pad):
    nchunks = src3.shape[1]
    rpt = n_pad // NS

    mesh = plsc.VectorSubcoreMesh(core_axis_name="c", subcore_axis_name="s")

    @functools.partial(
        pl.kernel,
        out_type=jax.ShapeDtypeStruct((NC, n_pad, 128), jnp.float32),
        mesh=mesh,
        scratch_types=[
            pltpu.VMEM((nchunks, CHUNK), jnp.int32),
            pltpu.VMEM((nchunks, CHUNK), jnp.int32),
            pltpu.VMEM((CHUNK, 128), jnp.float32),
            pltpu.VMEM((CHUNK, 128), jnp.float32),
            pltpu.VMEM_SHARED((n_pad, 128), jnp.float32),
            pltpu.SemaphoreType.DMA,
        ],
    )
    def k(g_hbm, src_hbm, dst_hbm, out_hbm, src_v, dst_v, buf_a, buf_b,
          acc_sh, sem):
        c = lax.axis_index("c")
        s = lax.axis_index("s")
        slab = c * NS + s
        zeros = jnp.zeros((16,), jnp.float32)

        def zbody(r, _):
            for u in range(8):
                buf_a[r, pl.ds(u * 16, 16)] = zeros
            return ()
        lax.fori_loop(0, CHUNK, zbody, ())
        for q in range(rpt // CHUNK):
            pltpu.sync_copy(buf_a, acc_sh.at[pl.ds(s * rpt + q * CHUNK, CHUNK)])

        pltpu.sync_copy(src_hbm.at[slab], src_v)
        pltpu.sync_copy(dst_hbm.at[slab], dst_v)
        plsc.subcore_barrier()

        def body(j, _):
            pltpu.async_copy(g_hbm.at[src_v.at[j]], buf_a, sem).wait()
            pltpu.sync_copy(buf_a, acc_sh.at[dst_v.at[j]], add=True)
            return ()
        lax.fori_loop(0, nchunks, body, ())
        plsc.subcore_barrier()

        for q in range(rpt // CHUNK):
            base = s * rpt + q * CHUNK
            pltpu.sync_copy(acc_sh.at[pl.ds(base, CHUNK)], buf_b)
            pltpu.sync_copy(buf_b, out_hbm.at[c, pl.ds(base, CHUNK)])

    return k(g, src3, dst3)


def _tc_first(x, w, d0, d1, n_pad):
    grid = (n_pad // BLK,)

    def body(x_ref, w_ref, d0_ref, d1_ref, g_ref, sl_ref):
        deg = d0_ref[...] + d1_ref[...] + 1.0
        dis = lax.rsqrt(deg)
        inv = 1.0 / deg
        h = jnp.dot(x_ref[...], w_ref[...], preferred_element_type=jnp.float32)
        g_ref[...] = h * dis
        sl_ref[...] = h * inv

    return pl.pallas_call(
        body,
        grid=grid,
        in_specs=[
            pl.BlockSpec((BLK, 128), lambda i: (i, 0)),
            pl.BlockSpec((128, 128), lambda i: (0, 0)),
            pl.BlockSpec((BLK, 1), lambda i: (i, 0)),
            pl.BlockSpec((BLK, 1), lambda i: (i, 0)),
        ],
        out_specs=[
            pl.BlockSpec((BLK, 128), lambda i: (i, 0)),
            pl.BlockSpec((BLK, 128), lambda i: (i, 0)),
        ],
        out_shape=[
            jax.ShapeDtypeStruct((n_pad, 128), jnp.float32),
            jax.ShapeDtypeStruct((n_pad, 128), jnp.float32),
        ],
    )(x, w, d0, d1)


def _tc_mid(sp, sl, b, w, d0, d1, n_pad):
    grid = (n_pad // BLK,)

    def body(sp_ref, sl_ref, b_ref, w_ref, d0_ref, d1_ref, g_ref, sl2_ref):
        deg = d0_ref[...] + d1_ref[...] + 1.0
        dis = lax.rsqrt(deg)
        inv = 1.0 / deg
        o = (sp_ref[0] + sp_ref[1]) * dis + sl_ref[...] + b_ref[...]
        h = jnp.dot(o, w_ref[...], preferred_element_type=jnp.float32)
        g_ref[...] = h * dis
        sl2_ref[...] = h * inv

    return pl.pallas_call(
        body,
        grid=grid,
        in_specs=[
            pl.BlockSpec((2, BLK, 128), lambda i: (0, i, 0)),
            pl.BlockSpec((BLK, 128), lambda i: (i, 0)),
            pl.BlockSpec((1, 128), lambda i: (0, 0)),
            pl.BlockSpec((128, 128), lambda i: (0, 0)),
            pl.BlockSpec((BLK, 1), lambda i: (i, 0)),
            pl.BlockSpec((BLK, 1), lambda i: (i, 0)),
        ],
        out_specs=[
            pl.BlockSpec((BLK, 128), lambda i: (i, 0)),
            pl.BlockSpec((BLK, 128), lambda i: (i, 0)),
        ],
        out_shape=[
            jax.ShapeDtypeStruct((n_pad, 128), jnp.float32),
            jax.ShapeDtypeStruct((n_pad, 128), jnp.float32),
        ],
    )(sp, sl, b, w, d0, d1)


def _tc_last(sp, sl, b, d0, d1, n_pad):
    grid = (n_pad // BLK,)

    def body(sp_ref, sl_ref, b_ref, d0_ref, d1_ref, o_ref):
        deg = d0_ref[...] + d1_ref[...] + 1.0
        dis = lax.rsqrt(deg)
        o_ref[...] = (sp_ref[0] + sp_ref[1]) * dis + sl_ref[...] + b_ref[...]

    return pl.pallas_call(
        body,
        grid=grid,
        in_specs=[
            pl.BlockSpec((2, BLK, 128), lambda i: (0, i, 0)),
            pl.BlockSpec((BLK, 128), lambda i: (i, 0)),
            pl.BlockSpec((1, 128), lambda i: (0, 0)),
            pl.BlockSpec((BLK, 1), lambda i: (i, 0)),
            pl.BlockSpec((BLK, 1), lambda i: (i, 0)),
        ],
        out_specs=pl.BlockSpec((BLK, 128), lambda i: (i, 0)),
        out_shape=jax.ShapeDtypeStruct((n_pad, 128), jnp.float32),
    )(sp, sl, b, d0, d1)


def kernel(x, edge_index, W1, b1, W2, b2):
    n, d = x.shape
    e = edge_index.shape[1]
    n_pad = _round_up(n + 1, BLK)

    src = edge_index[0].astype(jnp.int32)
    dst = edge_index[1].astype(jnp.int32)

    per_tile = _round_up(_round_up(e, NW) // NW, CHUNK)
    e_pad = per_tile * NW
    src = jnp.pad(src, (0, e_pad - e))
    dst = jnp.pad(dst, (0, e_pad - e), constant_values=n)
    src3 = src.reshape(NW, per_tile // CHUNK, CHUNK)
    dst3 = dst.reshape(NW, per_tile // CHUNK, CHUNK)

    x_pad = jnp.pad(x, ((0, n_pad - n), (0, 0)))
    b1r = b1.reshape(1, 128)
    b2r = b2.reshape(1, 128)

    deg_p = _sc_degree(dst3, n_pad)
    d0 = deg_p[0].reshape(n_pad, 1)
    d1 = deg_p[1].reshape(n_pad, 1)

    g1, sl1 = _tc_first(x_pad, W1, d0, d1, n_pad)
    sp1 = _sc_scatter(g1, src3, dst3, n_pad)
    g2, sl2 = _tc_mid(sp1, sl1, b1r, W2, d0, d1, n_pad)
    sp2 = _sc_scatter(g2, src3, dst3, n_pad)
    out = _tc_last(sp2, sl2, b2r, d0, d1, n_pad)
    return out[:n]


# speedup vs baseline: 13.4167x; 13.4167x over previous
import functools

import jax
import jax.numpy as jnp
from jax import lax
from jax.experimental import pallas as pl
from jax.experimental.pallas import tpu as pltpu
from jax.experimental.pallas import tpu_sc as plsc

NC = 2
NS = 16
NW = NC * NS
CHUNK = 128
BLK = 1024


def _round_up(a, b):
    return (a + b - 1) // b * b


def _sc_degree(dst3, n_pad):
    nchunks = dst3.shape[1]
    rpt = n_pad // NS

    mesh = plsc.VectorSubcoreMesh(core_axis_name="c", subcore_axis_name="s")

    @functools.partial(
        pl.kernel,
        out_type=jax.ShapeDtypeStruct((NC, n_pad), jnp.float32),
        mesh=mesh,
        scratch_types=[
            pltpu.VMEM((nchunks, CHUNK), jnp.int32),
            pltpu.VMEM((CHUNK,), jnp.float32),
            pltpu.VMEM((rpt,), jnp.float32),
            pltpu.VMEM_SHARED((n_pad,), jnp.float32),
        ],
    )
    def k(dst_hbm, deg_hbm, dst_v, ones_v, stage_v, deg_sh):
        c = lax.axis_index("c")
        s = lax.axis_index("s")
        slab = c * NS + s
        ones = jnp.ones((16,), jnp.float32)
        zeros = jnp.zeros((16,), jnp.float32)
        for u in range(CHUNK // 16):
            ones_v[pl.ds(u * 16, 16)] = ones

        def zbody(r, _):
            stage_v[pl.ds(r * 16, 16)] = zeros
            return ()
        lax.fori_loop(0, rpt // 16, zbody, ())
        pltpu.sync_copy(stage_v, deg_sh.at[pl.ds(s * rpt, rpt)])
        plsc.subcore_barrier()

        pltpu.sync_copy(dst_hbm.at[slab], dst_v)

        def body(j, _):
            pltpu.sync_copy(ones_v, deg_sh.at[dst_v.at[j]], add=True)
            return ()
        lax.fori_loop(0, nchunks, body, ())
        plsc.subcore_barrier()

        pltpu.sync_copy(deg_sh.at[pl.ds(s * rpt, rpt)], stage_v)
        pltpu.sync_copy(stage_v, deg_hbm.at[c, pl.ds(s * rpt, rpt)])

    return k(dst3)


def _sc_scatter(g, src3, dst3, n_pad):
    nchunks = src3.shape[1]
    rpt = n_pad // NS

    mesh = plsc.VectorSubcoreMesh(core_axis_name="c", subcore_axis_name="s")

    @functools.partial(
        pl.kernel,
        out_type=jax.ShapeDtypeStruct((NC, n_pad, 128), jnp.float32),
        mesh=mesh,
        scratch_types=[
            pltpu.VMEM((nchunks, CHUNK), jnp.int32),
            pltpu.VMEM((nchunks, CHUNK), jnp.int32),
            pltpu.VMEM((CHUNK, 128), jnp.float32),
            pltpu.VMEM_SHARED((n_pad, 128), jnp.float32),
            pltpu.SemaphoreType.DMA,
        ],
    )
    def k(g_hbm, src_hbm, dst_hbm, out_hbm, src_v, dst_v, buf_a,
          acc_sh, sem):
        c = lax.axis_index("c")
        s = lax.axis_index("s")
        slab = c * NS + s
        zeros = jnp.zeros((16,), jnp.float32)

        def zbody(r, _):
            for u in range(8):
                buf_a[r, pl.ds(u * 16, 16)] = zeros
            return ()
        lax.fori_loop(0, CHUNK, zbody, ())
        for q in range(rpt // CHUNK):
            pltpu.sync_copy(buf_a, acc_sh.at[pl.ds(s * rpt + q * CHUNK, CHUNK)])

        pltpu.sync_copy(src_hbm.at[slab], src_v)
        pltpu.sync_copy(dst_hbm.at[slab], dst_v)
        plsc.subcore_barrier()

        def body(j, _):
            pltpu.async_copy(g_hbm.at[src_v.at[j]], buf_a, sem).wait()
            pltpu.sync_copy(buf_a, acc_sh.at[dst_v.at[j]], add=True)
            return ()
        lax.fori_loop(0, nchunks, body, ())
        plsc.subcore_barrier()

        for q in range(pl.cdiv(rpt, CHUNK)):
            base = s * rpt + q * CHUNK
            rows = min(CHUNK, rpt - q * CHUNK)
            pltpu.sync_copy(acc_sh.at[pl.ds(base, rows)],
                            buf_a.at[pl.ds(0, rows)])
            pltpu.sync_copy(buf_a.at[pl.ds(0, rows)],
                            out_hbm.at[c, pl.ds(base, rows)])

    return k(g, src3, dst3)


def _tc_first(x, w, d0, d1, n_pad):
    grid = (n_pad // BLK,)

    def body(x_ref, w_ref, d0_ref, d1_ref, g_ref, sl_ref):
        deg = d0_ref[...] + d1_ref[...] + 1.0
        dis = lax.rsqrt(deg)
        inv = 1.0 / deg
        h = jnp.dot(x_ref[...], w_ref[...], preferred_element_type=jnp.float32)
        g_ref[...] = h * dis
        sl_ref[...] = h * inv

    return pl.pallas_call(
        body,
        grid=grid,
        in_specs=[
            pl.BlockSpec((BLK, 128), lambda i: (i, 0)),
            pl.BlockSpec((128, 128), lambda i: (0, 0)),
            pl.BlockSpec((BLK, 1), lambda i: (i, 0)),
            pl.BlockSpec((BLK, 1), lambda i: (i, 0)),
        ],
        out_specs=[
            pl.BlockSpec((BLK, 128), lambda i: (i, 0)),
            pl.BlockSpec((BLK, 128), lambda i: (i, 0)),
        ],
        out_shape=[
            jax.ShapeDtypeStruct((n_pad, 128), jnp.float32),
            jax.ShapeDtypeStruct((n_pad, 128), jnp.float32),
        ],
    )(x, w, d0, d1)


def _tc_mid(sp, sl, b, w, d0, d1, n_pad):
    grid = (n_pad // BLK,)

    def body(sp_ref, sl_ref, b_ref, w_ref, d0_ref, d1_ref, g_ref, sl2_ref):
        deg = d0_ref[...] + d1_ref[...] + 1.0
        dis = lax.rsqrt(deg)
        inv = 1.0 / deg
        o = (sp_ref[0] + sp_ref[1]) * dis + sl_ref[...] + b_ref[...]
        h = jnp.dot(o, w_ref[...], preferred_element_type=jnp.float32)
        g_ref[...] = h * dis
        sl2_ref[...] = h * inv

    return pl.pallas_call(
        body,
        grid=grid,
        in_specs=[
            pl.BlockSpec((2, BLK, 128), lambda i: (0, i, 0)),
            pl.BlockSpec((BLK, 128), lambda i: (i, 0)),
            pl.BlockSpec((1, 128), lambda i: (0, 0)),
            pl.BlockSpec((128, 128), lambda i: (0, 0)),
            pl.BlockSpec((BLK, 1), lambda i: (i, 0)),
            pl.BlockSpec((BLK, 1), lambda i: (i, 0)),
        ],
        out_specs=[
            pl.BlockSpec((BLK, 128), lambda i: (i, 0)),
            pl.BlockSpec((BLK, 128), lambda i: (i, 0)),
        ],
        out_shape=[
            jax.ShapeDtypeStruct((n_pad, 128), jnp.float32),
            jax.ShapeDtypeStruct((n_pad, 128), jnp.float32),
        ],
    )(sp, sl, b, w, d0, d1)


def _tc_last(sp, sl, b, d0, d1, n_pad):
    grid = (n_pad // BLK,)

    def body(sp_ref, sl_ref, b_ref, d0_ref, d1_ref, o_ref):
        deg = d0_ref[...] + d1_ref[...] + 1.0
        dis = lax.rsqrt(deg)
        o_ref[...] = (sp_ref[0] + sp_ref[1]) * dis + sl_ref[...] + b_ref[...]

    return pl.pallas_call(
        body,
        grid=grid,
        in_specs=[
            pl.BlockSpec((2, BLK, 128), lambda i: (0, i, 0)),
            pl.BlockSpec((BLK, 128), lambda i: (i, 0)),
            pl.BlockSpec((1, 128), lambda i: (0, 0)),
            pl.BlockSpec((BLK, 1), lambda i: (i, 0)),
            pl.BlockSpec((BLK, 1), lambda i: (i, 0)),
        ],
        out_specs=pl.BlockSpec((BLK, 128), lambda i: (i, 0)),
        out_shape=jax.ShapeDtypeStruct((n_pad, 128), jnp.float32),
    )(sp, sl, b, d0, d1)


def kernel(x, edge_index, W1, b1, W2, b2):
    n, d = x.shape
    e = edge_index.shape[1]
    n_pad = _round_up(n + 1, BLK)

    src = edge_index[0].astype(jnp.int32)
    dst = edge_index[1].astype(jnp.int32)

    per_tile = _round_up(_round_up(e, NW) // NW, CHUNK)
    e_pad = per_tile * NW
    src = jnp.pad(src, (0, e_pad - e))
    dst = jnp.pad(dst, (0, e_pad - e), constant_values=n)
    src3 = src.reshape(NW, per_tile // CHUNK, CHUNK)
    dst3 = dst.reshape(NW, per_tile // CHUNK, CHUNK)

    x_pad = jnp.pad(x, ((0, n_pad - n), (0, 0)))
    b1r = b1.reshape(1, 128)
    b2r = b2.reshape(1, 128)

    deg_p = _sc_degree(dst3, n_pad)
    d0 = deg_p[0].reshape(n_pad, 1)
    d1 = deg_p[1].reshape(n_pad, 1)

    g1, sl1 = _tc_first(x_pad, W1, d0, d1, n_pad)
    sp1 = _sc_scatter(g1, src3, dst3, n_pad)
    g2, sl2 = _tc_mid(sp1, sl1, b1r, W2, d0, d1, n_pad)
    sp2 = _sc_scatter(g2, src3, dst3, n_pad)
    out = _tc_last(sp2, sl2, b2r, d0, d1, n_pad)
    return out[:n]
